# baseline (speedup 1.0000x reference)
import dataclasses
import functools

import jax
import jax.numpy as jnp
from jax import lax
from jax.experimental import pallas as pl
from jax.experimental.pallas import tpu as pltpu
from jax.experimental.pallas import tpu_sc as plsc

_W = 0.01
_NC, _NS = 2, 16
_NW = _NC * _NS
_RB = 16


def _make_sc_stats(rows, cols):
    nr_w = rows // _NW
    nblk = nr_w // _RB
    mesh = plsc.VectorSubcoreMesh(
        core_axis_name="c", subcore_axis_name="s",
        num_cores=_NC, num_subcores=_NS,
    )

    cp = pltpu.CompilerParams()
    if "needs_layout_passes" in pltpu.CompilerParams.__dataclass_fields__:
        cp = dataclasses.replace(cp, needs_layout_passes=False)

    @functools.partial(
        pl.kernel,
        compiler_params=cp,
        out_type=[
            jax.ShapeDtypeStruct((rows,), jnp.float32),
            jax.ShapeDtypeStruct((rows,), jnp.float32),
            jax.ShapeDtypeStruct((rows,), jnp.float32),
        ],
        mesh=mesh,
        scratch_types=[
            pltpu.VMEM((_RB, cols), jnp.float32),
            pltpu.VMEM((_RB, cols), jnp.float32),
            pltpu.VMEM((_RB, cols), jnp.int32),
            pltpu.VMEM((_RB, cols), jnp.int32),
            pltpu.VMEM((nr_w,), jnp.float32),
            pltpu.VMEM((nr_w,), jnp.float32),
            pltpu.VMEM((nr_w,), jnp.float32),
            pltpu.SemaphoreType.DMA((2, 2)),
        ],
    )
    def sc_stats(x_hbm, t_hbm, d_hbm, s_hbm, c_hbm,
                 xb0, xb1, tb0, tb1, dst, sst, cst, sems):
        w = lax.axis_index("s") * _NC + lax.axis_index("c")
        base = w * nr_w
        xbufs = (xb0, xb1)
        tbufs = (tb0, tb1)

        def issue(k, slot):
            pltpu.async_copy(
                x_hbm.at[pl.ds(base + k * _RB, _RB)], xbufs[slot],
                sems.at[slot, 0])
            pltpu.async_copy(
                t_hbm.at[pl.ds(base + k * _RB, _RB)], tbufs[slot],
                sems.at[slot, 1])

        def wait_block(k, slot):
            pltpu.make_async_copy(
                x_hbm.at[pl.ds(base + k * _RB, _RB)], xbufs[slot],
                sems.at[slot, 0]).wait()
            pltpu.make_async_copy(
                t_hbm.at[pl.ds(base + k * _RB, _RB)], tbufs[slot],
                sems.at[slot, 1]).wait()

        issue(0, 0)
        issue(1, 1)
        row_i = lax.iota(jnp.int32, 16)
        zero = jnp.zeros((16,), jnp.float32)
        izero = jnp.zeros((16,), jnp.int32)
        tailmask = row_i >= 8

        nfull = cols // 16
        npair = nfull // 2

        def block(k, slot):
            wait_block(k, slot)
            xf = xbufs[slot]
            tf = tbufs[slot]
            dvec = zero
            svec = zero
            cvec = zero
            for r in range(_RB):
                rsplat = jnp.full((16,), r, jnp.int32)

                def jstep(j, carry, rsplat=rsplat):
                    d0, d1, s0, s1, c0, c1 = carry
                    off = j * 32
                    col0 = jnp.full((16,), off, jnp.int32) + row_i
                    col1 = jnp.full((16,), off + 16, jnp.int32) + row_i
                    xv0 = plsc.load_gather(xf, [rsplat, col0])
                    tv0 = plsc.load_gather(tf, [rsplat, col0])
                    m0 = tv0 != 0
                    xm0 = jnp.where(m0, xv0, -10000.0)
                    e0 = jnp.exp(xm0)
                    xv1 = plsc.load_gather(xf, [rsplat, col1])
                    tv1 = plsc.load_gather(tf, [rsplat, col1])
                    m1 = tv1 != 0
                    xm1 = jnp.where(m1, xv1, -10000.0)
                    e1 = jnp.exp(xm1)
                    return (d0 + e0, d1 + e1, s0 + e0 * xm0, s1 + e1 * xm1,
                            c0 + tv0, c1 + tv1)

                d0, d1, s0, s1, c0, c1 = lax.fori_loop(
                    0, npair, jstep, (zero, zero, zero, zero, izero, izero))
                ctail = jnp.full((16,), cols - 16, jnp.int32) + row_i
                xv = plsc.load_gather(xf, [rsplat, ctail])
                tv = plsc.load_gather(tf, [rsplat, ctail])
                m = jnp.logical_and(tv != 0, tailmask)
                xm = jnp.where(m, xv, -10000.0)
                e = jnp.exp(xm)
                d = (d0 + d1) + e
                s = (s0 + s1) + e * xm
                cf = (c0 + c1).astype(jnp.float32) + jnp.where(m, 1.0, 0.0)
                dvec = jnp.where(row_i == r, jnp.sum(d), dvec)
                svec = jnp.where(row_i == r, jnp.sum(s), svec)
                cvec = jnp.where(row_i == r, jnp.sum(cf), cvec)

            dst[pl.ds(k * _RB, _RB)] = dvec
            sst[pl.ds(k * _RB, _RB)] = svec
            cst[pl.ds(k * _RB, _RB)] = cvec

            @pl.when(k + 2 < nblk)
            def _():
                issue(k + 2, slot)

        def pairstep(p, _):
            for slot in (0, 1):
                block(p * 2 + slot, slot)
            return 0

        lax.fori_loop(0, nblk // 2, pairstep, 0)

        pltpu.sync_copy(dst, d_hbm.at[pl.ds(base, nr_w)])
        pltpu.sync_copy(sst, s_hbm.at[pl.ds(base, nr_w)])
        pltpu.sync_copy(cst, c_hbm.at[pl.ds(base, nr_w)])

    return sc_stats


def _combine_body(d_ref, s_ref, c_ref, out_ref):
    d = d_ref[...]
    s = s_ref[...]
    c = c_ref[...]
    dsafe = jnp.where(c > 0.0, d, 1.0)
    contrib = jnp.where(c > 0.0, jnp.log(dsafe) - s / dsafe, 0.0)
    out_ref[0, 0] = _W * jnp.sum(contrib) / jnp.sum(c)


def kernel(logits, target):
    rows, cols = logits.shape
    d, s, c = _make_sc_stats(rows, cols)(logits, target)
    side = 128
    d2 = d.reshape(rows // side, side)
    s2 = s.reshape(rows // side, side)
    c2 = c.reshape(rows // side, side)
    out = pl.pallas_call(
        _combine_body,
        out_specs=pl.BlockSpec(memory_space=pltpu.SMEM),
        out_shape=jax.ShapeDtypeStruct((1, 1), jnp.float32),
    )(d2, s2, c2)
    return out[0, 0]


# speedup vs baseline: 1.3451x; 1.3451x over previous
import dataclasses
import functools

import jax
import jax.numpy as jnp
from jax import lax
from jax.experimental import pallas as pl
from jax.experimental.pallas import tpu as pltpu
from jax.experimental.pallas import tpu_sc as plsc

_W = 0.01
_RT = 10240
_CR = 512
_NBUF = 4
_NC, _NS = 2, 16
_NW = _NC * _NS
_RB = 16


def _tc_chunk_stats(x, t):
    mask = t != 0
    xm = jnp.where(mask, x, -10000.0)
    m = jnp.max(xm, axis=1, keepdims=True)
    z = xm - m
    e = jnp.exp(z)
    d = jnp.sum(e, axis=1, keepdims=True)
    s = jnp.sum(e * z, axis=1, keepdims=True)
    cnt = jnp.sum(mask.astype(jnp.float32), axis=1, keepdims=True)
    dsafe = jnp.where(cnt > 0.0, d, 1.0)
    contrib = jnp.where(cnt > 0.0, jnp.log(dsafe) - s / dsafe, 0.0)
    return jnp.sum(contrib), jnp.sum(cnt)


def _tc_body(x_hbm, t_hbm, out_ref, xb, tb, sems):
    nchunks = _RT // _CR
    ngroups = nchunks // _NBUF

    def _issue(c, slot):
        pltpu.make_async_copy(
            x_hbm.at[pl.ds(c * _CR, _CR)], xb.at[slot], sems.at[slot, 0]
        ).start(priority=slot % 2)
        pltpu.make_async_copy(
            t_hbm.at[pl.ds(c * _CR, _CR)], tb.at[slot], sems.at[slot, 1]
        ).start(priority=(slot + 1) % 2)

    for c in range(_NBUF):
        _issue(c, c)

    def _group(g, carry):
        acc_s, acc_n = carry
        for b in range(_NBUF):
            c = g * _NBUF + b
            pltpu.make_async_copy(
                x_hbm.at[pl.ds(c * _CR, _CR)], xb.at[b], sems.at[b, 0]
            ).wait()
            pltpu.make_async_copy(
                t_hbm.at[pl.ds(c * _CR, _CR)], tb.at[b], sems.at[b, 1]
            ).wait()
            ds, dn = _tc_chunk_stats(xb[b], tb[b])

            @pl.when(c + _NBUF < nchunks)
            def _():
                _issue(c + _NBUF, b)

            acc_s, acc_n = acc_s + ds, acc_n + dn
        return acc_s, acc_n

    acc_s, acc_n = lax.fori_loop(0, ngroups, _group, (0.0, 0.0))
    out_ref[0, 0] = acc_s
    out_ref[0, 1] = acc_n


def _tc_part(logits, target):
    cols = logits.shape[1]
    return pl.pallas_call(
        _tc_body,
        in_specs=[
            pl.BlockSpec(memory_space=pl.ANY),
            pl.BlockSpec(memory_space=pl.ANY),
        ],
        out_specs=pl.BlockSpec(memory_space=pltpu.SMEM),
        out_shape=jax.ShapeDtypeStruct((1, 2), jnp.float32),
        scratch_shapes=[
            pltpu.VMEM((_NBUF, _CR, cols), jnp.float32),
            pltpu.VMEM((_NBUF, _CR, cols), jnp.int32),
            pltpu.SemaphoreType.DMA((_NBUF, 2)),
        ],
    )(logits, target)


def _make_sc_stats(rows, cols):
    sc_rows = rows - _RT
    nr_w = sc_rows // _NW
    nblk = nr_w // _RB
    mesh = plsc.VectorSubcoreMesh(
        core_axis_name="c", subcore_axis_name="s",
        num_cores=_NC, num_subcores=_NS,
    )
    cp = pltpu.CompilerParams()
    if "needs_layout_passes" in pltpu.CompilerParams.__dataclass_fields__:
        cp = dataclasses.replace(cp, needs_layout_passes=False)

    @functools.partial(
        pl.kernel,
        compiler_params=cp,
        out_type=[
            jax.ShapeDtypeStruct((sc_rows,), jnp.float32),
            jax.ShapeDtypeStruct((sc_rows,), jnp.float32),
            jax.ShapeDtypeStruct((sc_rows,), jnp.float32),
        ],
        mesh=mesh,
        scratch_types=[
            pltpu.VMEM((_RB, cols), jnp.float32),
            pltpu.VMEM((_RB, cols), jnp.float32),
            pltpu.VMEM((_RB, cols), jnp.int32),
            pltpu.VMEM((_RB, cols), jnp.int32),
            pltpu.VMEM((nr_w,), jnp.float32),
            pltpu.VMEM((nr_w,), jnp.float32),
            pltpu.VMEM((nr_w,), jnp.float32),
            pltpu.SemaphoreType.DMA((2, 2)),
        ],
    )
    def sc_stats(x_hbm, t_hbm, d_hbm, s_hbm, c_hbm,
                 xb0, xb1, tb0, tb1, dst, sst, cst, sems):
        w = lax.axis_index("s") * _NC + lax.axis_index("c")
        base = _RT + w * nr_w
        xbufs = (xb0, xb1)
        tbufs = (tb0, tb1)

        def issue(k, slot):
            pltpu.async_copy(
                x_hbm.at[pl.ds(base + k * _RB, _RB)], xbufs[slot],
                sems.at[slot, 0])
            pltpu.async_copy(
                t_hbm.at[pl.ds(base + k * _RB, _RB)], tbufs[slot],
                sems.at[slot, 1])

        def wait_block(k, slot):
            pltpu.make_async_copy(
                x_hbm.at[pl.ds(base + k * _RB, _RB)], xbufs[slot],
                sems.at[slot, 0]).wait()
            pltpu.make_async_copy(
                t_hbm.at[pl.ds(base + k * _RB, _RB)], tbufs[slot],
                sems.at[slot, 1]).wait()

        issue(0, 0)
        issue(1, 1)
        row_i = lax.iota(jnp.int32, 16)
        zero = jnp.zeros((16,), jnp.float32)
        izero = jnp.zeros((16,), jnp.int32)
        tailmask = row_i >= 8

        npair = (cols // 16) // 2

        def block(k, slot):
            wait_block(k, slot)
            xf = xbufs[slot]
            tf = tbufs[slot]
            dvec = zero
            svec = zero
            cvec = zero
            for r in range(_RB):
                rsplat = jnp.full((16,), r, jnp.int32)

                def jstep(j, carry, rsplat=rsplat, xf=xf, tf=tf):
                    d0, d1, s0, s1, c0, c1 = carry
                    off = j * 32
                    col0 = jnp.full((16,), off, jnp.int32) + row_i
                    col1 = jnp.full((16,), off + 16, jnp.int32) + row_i
                    xv0 = plsc.load_gather(xf, [rsplat, col0])
                    tv0 = plsc.load_gather(tf, [rsplat, col0])
                    m0 = tv0 != 0
                    xm0 = jnp.where(m0, xv0, -10000.0)
                    e0 = jnp.exp(xm0)
                    xv1 = plsc.load_gather(xf, [rsplat, col1])
                    tv1 = plsc.load_gather(tf, [rsplat, col1])
                    m1 = tv1 != 0
                    xm1 = jnp.where(m1, xv1, -10000.0)
                    e1 = jnp.exp(xm1)
                    return (d0 + e0, d1 + e1, s0 + e0 * xm0, s1 + e1 * xm1,
                            c0 + tv0, c1 + tv1)

                d0, d1, s0, s1, c0, c1 = lax.fori_loop(
                    0, npair, jstep, (zero, zero, zero, zero, izero, izero))
                ctail = jnp.full((16,), cols - 16, jnp.int32) + row_i
                xv = plsc.load_gather(xf, [rsplat, ctail])
                tv = plsc.load_gather(tf, [rsplat, ctail])
                m = jnp.logical_and(tv != 0, tailmask)
                xm = jnp.where(m, xv, -10000.0)
                e = jnp.exp(xm)
                d = (d0 + d1) + e
                s = (s0 + s1) + e * xm
                cf = (c0 + c1).astype(jnp.float32) + jnp.where(m, 1.0, 0.0)
                dvec = jnp.where(row_i == r, jnp.sum(d), dvec)
                svec = jnp.where(row_i == r, jnp.sum(s), svec)
                cvec = jnp.where(row_i == r, jnp.sum(cf), cvec)

            dst[pl.ds(k * _RB, _RB)] = dvec
            sst[pl.ds(k * _RB, _RB)] = svec
            cst[pl.ds(k * _RB, _RB)] = cvec

            @pl.when(k + 2 < nblk)
            def _():
                issue(k + 2, slot)

        def pairstep(p, _):
            for slot in (0, 1):
                block(p * 2 + slot, slot)
            return 0

        lax.fori_loop(0, nblk // 2, pairstep, 0)

        pltpu.sync_copy(dst, d_hbm.at[pl.ds(w * nr_w, nr_w)])
        pltpu.sync_copy(sst, s_hbm.at[pl.ds(w * nr_w, nr_w)])
        pltpu.sync_copy(cst, c_hbm.at[pl.ds(w * nr_w, nr_w)])

    return sc_stats


def _combine_body(d_ref, s_ref, c_ref, p_ref, out_ref):
    d = d_ref[...]
    s = s_ref[...]
    c = c_ref[...]
    dsafe = jnp.where(c > 0.0, d, 1.0)
    contrib = jnp.where(c > 0.0, jnp.log(dsafe) - s / dsafe, 0.0)
    tot = jnp.sum(contrib) + p_ref[0, 0]
    n = jnp.sum(c) + p_ref[0, 1]
    out_ref[0, 0] = _W * tot / n


def kernel(logits, target):
    rows, cols = logits.shape
    part = _tc_part(logits, target)
    d, s, c = _make_sc_stats(rows, cols)(logits, target)
    side = 128
    nrow = (rows - _RT) // side
    d2 = d.reshape(nrow, side)
    s2 = s.reshape(nrow, side)
    c2 = c.reshape(nrow, side)
    out = pl.pallas_call(
        _combine_body,
        in_specs=[
            pl.BlockSpec((nrow, side), lambda: (0, 0)),
            pl.BlockSpec((nrow, side), lambda: (0, 0)),
            pl.BlockSpec((nrow, side), lambda: (0, 0)),
            pl.BlockSpec(memory_space=pltpu.SMEM),
        ],
        out_specs=pl.BlockSpec(memory_space=pltpu.SMEM),
        out_shape=jax.ShapeDtypeStruct((1, 1), jnp.float32),
    )(d2, s2, c2, part)
    return out[0, 0]


# speedup vs baseline: 1.3463x; 1.0009x over previous
import dataclasses
import functools

import jax
import jax.numpy as jnp
from jax import lax
from jax.experimental import pallas as pl
from jax.experimental.pallas import tpu as pltpu
from jax.experimental.pallas import tpu_sc as plsc

_W = 0.01
_RT = 10240
_CR = 512
_NBUF = 4
_NC, _NS = 2, 16
_NW = _NC * _NS
_RB = 16


def _tc_chunk_stats(x, t):
    mask = t != 0
    xm = jnp.where(mask, x, -10000.0)
    m = jnp.max(xm, axis=1, keepdims=True)
    z = xm - m
    e = jnp.exp(z)
    d = jnp.sum(e, axis=1, keepdims=True)
    s = jnp.sum(e * z, axis=1, keepdims=True)
    cnt = jnp.sum(mask.astype(jnp.float32), axis=1, keepdims=True)
    dsafe = jnp.where(cnt > 0.0, d, 1.0)
    contrib = jnp.where(cnt > 0.0, jnp.log(dsafe) - s / dsafe, 0.0)
    return jnp.sum(contrib), jnp.sum(cnt)


def _tc_body(x_hbm, t_hbm, out_ref, xb, tb, sems):
    nchunks = _RT // _CR
    ngroups = nchunks // _NBUF

    def _issue(c, slot):
        pltpu.make_async_copy(
            x_hbm.at[pl.ds(c * _CR, _CR)], xb.at[slot], sems.at[slot, 0]
        ).start(priority=slot % 2)
        pltpu.make_async_copy(
            t_hbm.at[pl.ds(c * _CR, _CR)], tb.at[slot], sems.at[slot, 1]
        ).start(priority=(slot + 1) % 2)

    for c in range(_NBUF):
        _issue(c, c)

    def _group(g, carry):
        acc_s, acc_n = carry
        for b in range(_NBUF):
            c = g * _NBUF + b
            pltpu.make_async_copy(
                x_hbm.at[pl.ds(c * _CR, _CR)], xb.at[b], sems.at[b, 0]
            ).wait()
            pltpu.make_async_copy(
                t_hbm.at[pl.ds(c * _CR, _CR)], tb.at[b], sems.at[b, 1]
            ).wait()
            ds, dn = _tc_chunk_stats(xb[b], tb[b])

            @pl.when(c + _NBUF < nchunks)
            def _():
                _issue(c + _NBUF, b)

            acc_s, acc_n = acc_s + ds, acc_n + dn
        return acc_s, acc_n

    acc_s, acc_n = lax.fori_loop(0, ngroups, _group, (0.0, 0.0))
    out_ref[0, 0] = acc_s
    out_ref[0, 1] = acc_n


def _tc_part(logits, target):
    cols = logits.shape[1]
    return pl.pallas_call(
        _tc_body,
        in_specs=[
            pl.BlockSpec(memory_space=pl.ANY),
            pl.BlockSpec(memory_space=pl.ANY),
        ],
        out_specs=pl.BlockSpec(memory_space=pltpu.SMEM),
        out_shape=jax.ShapeDtypeStruct((1, 2), jnp.float32),
        scratch_shapes=[
            pltpu.VMEM((_NBUF, _CR, cols), jnp.float32),
            pltpu.VMEM((_NBUF, _CR, cols), jnp.int32),
            pltpu.SemaphoreType.DMA((_NBUF, 2)),
        ],
    )(logits, target)


def _make_sc_stats(rows, cols):
    sc_rows = rows - _RT
    nr_w = sc_rows // _NW
    nblk = nr_w // _RB
    mesh = plsc.VectorSubcoreMesh(
        core_axis_name="c", subcore_axis_name="s",
        num_cores=_NC, num_subcores=_NS,
    )
    cp = pltpu.CompilerParams()
    if "needs_layout_passes" in pltpu.CompilerParams.__dataclass_fields__:
        cp = dataclasses.replace(cp, needs_layout_passes=False)

    @functools.partial(
        pl.kernel,
        compiler_params=cp,
        out_type=[
            jax.ShapeDtypeStruct((sc_rows,), jnp.float32),
            jax.ShapeDtypeStruct((sc_rows,), jnp.float32),
            jax.ShapeDtypeStruct((sc_rows,), jnp.float32),
        ],
        mesh=mesh,
        scratch_types=[
            pltpu.VMEM((_RB, cols), jnp.float32),
            pltpu.VMEM((_RB, cols), jnp.float32),
            pltpu.VMEM((_RB, cols), jnp.int32),
            pltpu.VMEM((_RB, cols), jnp.int32),
            pltpu.VMEM((nr_w,), jnp.float32),
            pltpu.VMEM((nr_w,), jnp.float32),
            pltpu.VMEM((nr_w,), jnp.float32),
            pltpu.SemaphoreType.DMA((2, 2)),
        ],
    )
    def sc_stats(x_hbm, t_hbm, d_hbm, s_hbm, c_hbm,
                 xb0, xb1, tb0, tb1, dst, sst, cst, sems):
        w = lax.axis_index("s") * _NC + lax.axis_index("c")
        base = _RT + w * nr_w
        xbufs = (xb0, xb1)
        tbufs = (tb0, tb1)

        def issue(k, slot):
            pltpu.async_copy(
                x_hbm.at[pl.ds(base + k * _RB, _RB)], xbufs[slot],
                sems.at[slot, 0])
            pltpu.async_copy(
                t_hbm.at[pl.ds(base + k * _RB, _RB)], tbufs[slot],
                sems.at[slot, 1])

        def wait_block(k, slot):
            pltpu.make_async_copy(
                x_hbm.at[pl.ds(base + k * _RB, _RB)], xbufs[slot],
                sems.at[slot, 0]).wait()
            pltpu.make_async_copy(
                t_hbm.at[pl.ds(base + k * _RB, _RB)], tbufs[slot],
                sems.at[slot, 1]).wait()

        issue(0, 0)
        issue(1, 1)
        row_i = lax.iota(jnp.int32, 16)
        zero = jnp.zeros((16,), jnp.float32)
        izero = jnp.zeros((16,), jnp.int32)
        tailmask = row_i >= 8

        npair = (cols // 16) // 2

        def block(k, slot):
            wait_block(k, slot)
            xf = xbufs[slot]
            tf = tbufs[slot]
            dvec = zero
            svec = zero
            cvec = zero
            for r in range(_RB):
                rsplat = jnp.full((16,), r, jnp.int32)

                def jstep(j, carry, rsplat=rsplat, xf=xf, tf=tf):
                    d0, d1, s0, s1, c0, c1 = carry
                    off = j * 32
                    col0 = jnp.full((16,), off, jnp.int32) + row_i
                    col1 = jnp.full((16,), off + 16, jnp.int32) + row_i
                    xv0 = plsc.load_gather(xf, [rsplat, col0])
                    tv0 = plsc.load_gather(tf, [rsplat, col0])
                    m0 = tv0 != 0
                    xm0 = jnp.where(m0, xv0, -10000.0)
                    e0 = jnp.exp(xm0)
                    xv1 = plsc.load_gather(xf, [rsplat, col1])
                    tv1 = plsc.load_gather(tf, [rsplat, col1])
                    m1 = tv1 != 0
                    xm1 = jnp.where(m1, xv1, -10000.0)
                    e1 = jnp.exp(xm1)
                    return (d0 + e0, d1 + e1, s0 + e0 * xm0, s1 + e1 * xm1,
                            c0 + tv0, c1 + tv1)

                d0, d1, s0, s1, c0, c1 = lax.fori_loop(
                    0, npair, jstep, (zero, zero, zero, zero, izero, izero))
                ctail = jnp.full((16,), cols - 16, jnp.int32) + row_i
                xv = plsc.load_gather(xf, [rsplat, ctail])
                tv = plsc.load_gather(tf, [rsplat, ctail])
                m = jnp.logical_and(tv != 0, tailmask)
                xm = jnp.where(m, xv, -10000.0)
                e = jnp.exp(xm)
                d = (d0 + d1) + e
                s = (s0 + s1) + e * xm
                cf = (c0 + c1).astype(jnp.float32) + jnp.where(m, 1.0, 0.0)
                dvec = jnp.where(row_i == r, jnp.sum(d), dvec)
                svec = jnp.where(row_i == r, jnp.sum(s), svec)
                cvec = jnp.where(row_i == r, jnp.sum(cf), cvec)

            dst[pl.ds(k * _RB, _RB)] = dvec
            sst[pl.ds(k * _RB, _RB)] = svec
            cst[pl.ds(k * _RB, _RB)] = cvec

            @pl.when(k + 2 < nblk)
            def _():
                issue(k + 2, slot)

        def pairstep(p, _):
            for slot in (0, 1):
                block(p * 2 + slot, slot)
            return 0

        lax.fori_loop(0, nblk // 2, pairstep, 0)

        pltpu.sync_copy(dst, d_hbm.at[pl.ds(w * nr_w, nr_w)])
        pltpu.sync_copy(sst, s_hbm.at[pl.ds(w * nr_w, nr_w)])
        pltpu.sync_copy(cst, c_hbm.at[pl.ds(w * nr_w, nr_w)])

    return sc_stats


def _combine_body(d_ref, s_ref, c_ref, p_ref, out_ref):
    d = d_ref[...]
    s = s_ref[...]
    c = c_ref[...]
    dsafe = jnp.where(c > 0.0, d, 1.0)
    contrib = jnp.where(c > 0.0, jnp.log(dsafe) - s / dsafe, 0.0)
    tot = jnp.sum(contrib) + p_ref[0, 0]
    n = jnp.sum(c) + p_ref[0, 1]
    out_ref[0, 0] = _W * tot / n


def kernel(logits, target):
    rows, cols = logits.shape
    d, s, c = _make_sc_stats(rows, cols)(logits, target)
    part = _tc_part(logits, target)
    side = 128
    nrow = (rows - _RT) // side
    d2 = d.reshape(nrow, side)
    s2 = s.reshape(nrow, side)
    c2 = c.reshape(nrow, side)
    out = pl.pallas_call(
        _combine_body,
        in_specs=[
            pl.BlockSpec((nrow, side), lambda: (0, 0)),
            pl.BlockSpec((nrow, side), lambda: (0, 0)),
            pl.BlockSpec((nrow, side), lambda: (0, 0)),
            pl.BlockSpec(memory_space=pltpu.SMEM),
        ],
        out_specs=pl.BlockSpec(memory_space=pltpu.SMEM),
        out_shape=jax.ShapeDtypeStruct((1, 1), jnp.float32),
    )(d2, s2, c2, part)
    return out[0, 0]


# speedup vs baseline: 1.4079x; 1.0458x over previous
import dataclasses
import functools

import jax
import jax.numpy as jnp
from jax import lax
from jax.experimental import pallas as pl
from jax.experimental.pallas import tpu as pltpu
from jax.experimental.pallas import tpu_sc as plsc

_W = 0.01
_RT = 14336
_CR = 512
_NBUF = 4
_NC, _NS = 2, 16
_NW = _NC * _NS
_RB = 16


def _tc_chunk_stats(x, t):
    mask = t != 0
    xm = jnp.where(mask, x, -10000.0)
    m = jnp.max(xm, axis=1, keepdims=True)
    z = xm - m
    e = jnp.exp(z)
    d = jnp.sum(e, axis=1, keepdims=True)
    s = jnp.sum(e * z, axis=1, keepdims=True)
    cnt = jnp.sum(mask.astype(jnp.float32), axis=1, keepdims=True)
    dsafe = jnp.where(cnt > 0.0, d, 1.0)
    contrib = jnp.where(cnt > 0.0, jnp.log(dsafe) - s / dsafe, 0.0)
    return jnp.sum(contrib), jnp.sum(cnt)


def _tc_body(x_hbm, t_hbm, out_ref, xb, tb, sems):
    nchunks = _RT // _CR
    ngroups = nchunks // _NBUF

    def _issue(c, slot):
        pltpu.make_async_copy(
            x_hbm.at[pl.ds(c * _CR, _CR)], xb.at[slot], sems.at[slot, 0]
        ).start(priority=slot % 2)
        pltpu.make_async_copy(
            t_hbm.at[pl.ds(c * _CR, _CR)], tb.at[slot], sems.at[slot, 1]
        ).start(priority=(slot + 1) % 2)

    for c in range(_NBUF):
        _issue(c, c)

    def _group(g, carry):
        acc_s, acc_n = carry
        for b in range(_NBUF):
            c = g * _NBUF + b
            pltpu.make_async_copy(
                x_hbm.at[pl.ds(c * _CR, _CR)], xb.at[b], sems.at[b, 0]
            ).wait()
            pltpu.make_async_copy(
                t_hbm.at[pl.ds(c * _CR, _CR)], tb.at[b], sems.at[b, 1]
            ).wait()
            ds, dn = _tc_chunk_stats(xb[b], tb[b])

            @pl.when(c + _NBUF < nchunks)
            def _():
                _issue(c + _NBUF, b)

            acc_s, acc_n = acc_s + ds, acc_n + dn
        return acc_s, acc_n

    acc_s, acc_n = lax.fori_loop(0, ngroups, _group, (0.0, 0.0))
    out_ref[0, 0] = acc_s
    out_ref[0, 1] = acc_n


def _tc_part(logits, target):
    cols = logits.shape[1]
    return pl.pallas_call(
        _tc_body,
        in_specs=[
            pl.BlockSpec(memory_space=pl.ANY),
            pl.BlockSpec(memory_space=pl.ANY),
        ],
        out_specs=pl.BlockSpec(memory_space=pltpu.SMEM),
        out_shape=jax.ShapeDtypeStruct((1, 2), jnp.float32),
        scratch_shapes=[
            pltpu.VMEM((_NBUF, _CR, cols), jnp.float32),
            pltpu.VMEM((_NBUF, _CR, cols), jnp.int32),
            pltpu.SemaphoreType.DMA((_NBUF, 2)),
        ],
    )(logits, target)


def _make_sc_stats(rows, cols):
    sc_rows = rows - _RT
    nr_w = sc_rows // _NW
    nblk = nr_w // _RB
    mesh = plsc.VectorSubcoreMesh(
        core_axis_name="c", subcore_axis_name="s",
        num_cores=_NC, num_subcores=_NS,
    )
    cp = pltpu.CompilerParams()
    if "needs_layout_passes" in pltpu.CompilerParams.__dataclass_fields__:
        cp = dataclasses.replace(cp, needs_layout_passes=False)

    @functools.partial(
        pl.kernel,
        compiler_params=cp,
        out_type=[
            jax.ShapeDtypeStruct((sc_rows,), jnp.float32),
            jax.ShapeDtypeStruct((sc_rows,), jnp.float32),
            jax.ShapeDtypeStruct((sc_rows,), jnp.float32),
        ],
        mesh=mesh,
        scratch_types=[
            pltpu.VMEM((_RB, cols), jnp.float32),
            pltpu.VMEM((_RB, cols), jnp.float32),
            pltpu.VMEM((_RB, cols), jnp.int32),
            pltpu.VMEM((_RB, cols), jnp.int32),
            pltpu.VMEM((nr_w,), jnp.float32),
            pltpu.VMEM((nr_w,), jnp.float32),
            pltpu.VMEM((nr_w,), jnp.float32),
            pltpu.SemaphoreType.DMA((2, 2)),
        ],
    )
    def sc_stats(x_hbm, t_hbm, d_hbm, s_hbm, c_hbm,
                 xb0, xb1, tb0, tb1, dst, sst, cst, sems):
        w = lax.axis_index("s") * _NC + lax.axis_index("c")
        base = _RT + w * nr_w
        xbufs = (xb0, xb1)
        tbufs = (tb0, tb1)

        def issue(k, slot):
            pltpu.async_copy(
                x_hbm.at[pl.ds(base + k * _RB, _RB)], xbufs[slot],
                sems.at[slot, 0])
            pltpu.async_copy(
                t_hbm.at[pl.ds(base + k * _RB, _RB)], tbufs[slot],
                sems.at[slot, 1])

        def wait_block(k, slot):
            pltpu.make_async_copy(
                x_hbm.at[pl.ds(base + k * _RB, _RB)], xbufs[slot],
                sems.at[slot, 0]).wait()
            pltpu.make_async_copy(
                t_hbm.at[pl.ds(base + k * _RB, _RB)], tbufs[slot],
                sems.at[slot, 1]).wait()

        issue(0, 0)
        issue(1, 1)
        row_i = lax.iota(jnp.int32, 16)
        zero = jnp.zeros((16,), jnp.float32)
        izero = jnp.zeros((16,), jnp.int32)
        tailmask = row_i >= 8

        npair = (cols // 16) // 2

        def block(k, slot):
            wait_block(k, slot)
            xf = xbufs[slot]
            tf = tbufs[slot]
            dvec = zero
            svec = zero
            cvec = zero
            for r in range(_RB):
                rsplat = jnp.full((16,), r, jnp.int32)

                def jstep(j, carry, rsplat=rsplat, xf=xf, tf=tf):
                    d0, d1, s0, s1, c0, c1 = carry
                    off = j * 32
                    col0 = jnp.full((16,), off, jnp.int32) + row_i
                    col1 = jnp.full((16,), off + 16, jnp.int32) + row_i
                    xv0 = plsc.load_gather(xf, [rsplat, col0])
                    tv0 = plsc.load_gather(tf, [rsplat, col0])
                    m0 = tv0 != 0
                    xm0 = jnp.where(m0, xv0, -10000.0)
                    e0 = jnp.exp(xm0)
                    xv1 = plsc.load_gather(xf, [rsplat, col1])
                    tv1 = plsc.load_gather(tf, [rsplat, col1])
                    m1 = tv1 != 0
                    xm1 = jnp.where(m1, xv1, -10000.0)
                    e1 = jnp.exp(xm1)
                    return (d0 + e0, d1 + e1, s0 + e0 * xm0, s1 + e1 * xm1,
                            c0 + tv0, c1 + tv1)

                d0, d1, s0, s1, c0, c1 = lax.fori_loop(
                    0, npair, jstep, (zero, zero, zero, zero, izero, izero))
                ctail = jnp.full((16,), cols - 16, jnp.int32) + row_i
                xv = plsc.load_gather(xf, [rsplat, ctail])
                tv = plsc.load_gather(tf, [rsplat, ctail])
                m = jnp.logical_and(tv != 0, tailmask)
                xm = jnp.where(m, xv, -10000.0)
                e = jnp.exp(xm)
                d = (d0 + d1) + e
                s = (s0 + s1) + e * xm
                cf = (c0 + c1).astype(jnp.float32) + jnp.where(m, 1.0, 0.0)
                dvec = jnp.where(row_i == r, jnp.sum(d), dvec)
                svec = jnp.where(row_i == r, jnp.sum(s), svec)
                cvec = jnp.where(row_i == r, jnp.sum(cf), cvec)

            dst[pl.ds(k * _RB, _RB)] = dvec
            sst[pl.ds(k * _RB, _RB)] = svec
            cst[pl.ds(k * _RB, _RB)] = cvec

            @pl.when(k + 2 < nblk)
            def _():
                issue(k + 2, slot)

        def pairstep(p, _):
            for slot in (0, 1):
                block(p * 2 + slot, slot)
            return 0

        lax.fori_loop(0, nblk // 2, pairstep, 0)

        pltpu.sync_copy(dst, d_hbm.at[pl.ds(w * nr_w, nr_w)])
        pltpu.sync_copy(sst, s_hbm.at[pl.ds(w * nr_w, nr_w)])
        pltpu.sync_copy(cst, c_hbm.at[pl.ds(w * nr_w, nr_w)])

    return sc_stats


def _combine_body(d_ref, s_ref, c_ref, p_ref, out_ref):
    d = d_ref[...]
    s = s_ref[...]
    c = c_ref[...]
    dsafe = jnp.where(c > 0.0, d, 1.0)
    contrib = jnp.where(c > 0.0, jnp.log(dsafe) - s / dsafe, 0.0)
    tot = jnp.sum(contrib) + p_ref[0, 0]
    n = jnp.sum(c) + p_ref[0, 1]
    out_ref[0, 0] = _W * tot / n


def kernel(logits, target):
    rows, cols = logits.shape
    d, s, c = _make_sc_stats(rows, cols)(logits, target)
    part = _tc_part(logits, target)
    side = 128
    nrow = (rows - _RT) // side
    d2 = d.reshape(nrow, side)
    s2 = s.reshape(nrow, side)
    c2 = c.reshape(nrow, side)
    out = pl.pallas_call(
        _combine_body,
        in_specs=[
            pl.BlockSpec((nrow, side), lambda: (0, 0)),
            pl.BlockSpec((nrow, side), lambda: (0, 0)),
            pl.BlockSpec((nrow, side), lambda: (0, 0)),
            pl.BlockSpec(memory_space=pltpu.SMEM),
        ],
        out_specs=pl.BlockSpec(memory_space=pltpu.SMEM),
        out_shape=jax.ShapeDtypeStruct((1, 1), jnp.float32),
    )(d2, s2, c2, part)
    return out[0, 0]
